# Initial kernel scaffold; baseline (speedup 1.0000x reference)
import jax
import jax.numpy as jnp
from jax.experimental import pallas as pl


def kernel(image_features, text_features, W_ti, b_ti, W_ii, b_ii, W_c, b_c, W_o, b_o, W_d1, b_d1, W_d2, b_d2, codebooks):
    raise NotImplementedError("write your pallas kernel here")


# speedup vs baseline: 1.2531x; 1.2531x over previous
import functools

import jax
import jax.numpy as jnp
from jax import lax
from jax.experimental import pallas as pl
from jax.experimental.pallas import tpu as pltpu
from jax.experimental.pallas import tpu_sc as plsc

_B = 1024
_D = 768
_P = 2560
_H = 5120
_K = 8192
_L = 4

_PT = 512
_HT = 256
_KT = 1024


def _dotT(a, b, precision=None):
    return lax.dot_general(a, b, (((1,), (1,)), ((), ())),
                           preferred_element_type=jnp.float32,
                           precision=precision)


def _raw_body(feat_ref, w_ref, b_ref, out_ref):
    f = feat_ref[0]
    out_ref[...] = jnp.maximum(_dotT(f, w_ref[...]) + b_ref[...],
                               0.0).astype(jnp.bfloat16)


def _raw_call(feats, w_cat, b_cat):
    n_per_half = _P // _PT
    return pl.pallas_call(
        _raw_body,
        grid=(2 * _P // _PT,),
        in_specs=[
            pl.BlockSpec((1, _B, _D), lambda j: (j // n_per_half, 0, 0)),
            pl.BlockSpec((_PT, _D), lambda j: (j, 0)),
            pl.BlockSpec((1, _PT), lambda j: (0, j)),
        ],
        out_specs=pl.BlockSpec((_B, _PT), lambda j: (0, j)),
        out_shape=jax.ShapeDtypeStruct((_B, 2 * _P), jnp.bfloat16),
    )(feats, w_cat, b_cat)


def _comb_body(raw_ref, wc_ref, bc_ref, wd1_ref, bd1_ref, comb_ref, d1_ref):
    r = raw_ref[...].astype(jnp.float32)
    comb_ref[...] = jnp.maximum(_dotT(r, wc_ref[...]) + bc_ref[...], 0.0)
    d1_ref[...] = jnp.maximum(_dotT(r, wd1_ref[...]) + bd1_ref[...], 0.0)


def _comb_call(raw, W_c, b_c, W_d1, b_d1):
    return pl.pallas_call(
        _comb_body,
        grid=(_H // _HT,),
        in_specs=[
            pl.BlockSpec((_B, 2 * _P), lambda j: (0, 0)),
            pl.BlockSpec((_HT, 2 * _P), lambda j: (j, 0)),
            pl.BlockSpec((1, _HT), lambda j: (0, j)),
            pl.BlockSpec((_HT, 2 * _P), lambda j: (j, 0)),
            pl.BlockSpec((1, _HT), lambda j: (0, j)),
        ],
        out_specs=[
            pl.BlockSpec((_B, _HT), lambda j: (0, j)),
            pl.BlockSpec((_B, _HT), lambda j: (0, j)),
        ],
        out_shape=[
            jax.ShapeDtypeStruct((_B, _H), jnp.float32),
            jax.ShapeDtypeStruct((_B, _H), jnp.float32),
        ],
    )(raw, W_c, b_c.reshape(1, -1), W_d1, b_d1.reshape(1, -1))


def _ds_body(d1_ref, w2_ref, b2_ref, ds_ref):
    mv = _dotT(d1_ref[...], w2_ref[...])[:, :1]
    ds_ref[...] = jax.nn.sigmoid(mv + b2_ref[0])


def _ds_call(d1, W_d2, b_d2):
    w2_pad = jnp.concatenate([W_d2, jnp.zeros((7, _H), jnp.float32)], axis=0)
    return pl.pallas_call(
        _ds_body,
        in_specs=[
            pl.BlockSpec(memory_space=pltpu.VMEM),
            pl.BlockSpec(memory_space=pltpu.VMEM),
            pl.BlockSpec(memory_space=pltpu.SMEM),
        ],
        out_shape=jax.ShapeDtypeStruct((_B, 1), jnp.float32),
    )(d1, w2_pad, b_d2)


def _x_body(comb_ref, wo_ref, bo_ref, ds_ref, t_ref, i_ref, x_ref):
    mm = _dotT(comb_ref[...], wo_ref[...])
    ds = ds_ref[...]
    x_ref[...] = ((mm + bo_ref[...]) + ds * t_ref[...]) + (1.0 - ds) * i_ref[...]


def _x_call(comb, W_o, b_o, ds, text, image):
    return pl.pallas_call(
        _x_body,
        out_shape=jax.ShapeDtypeStruct((_B, _D), jnp.float32),
    )(comb, W_o, b_o.reshape(1, -1), ds, text, image)


def _dist_body(level, resprev_ref, qprev_ref, cb_ref,
               idxl_ref, idxg_ref, res_ref, rn_s, bv_s, bi_s):
    j = pl.program_id(0)

    @pl.when(j == 0)
    def _():
        r0 = resprev_ref[...] - qprev_ref[...]
        res_ref[...] = r0
        rn_s[...] = jnp.sum(r0 * r0, axis=1, keepdims=True)
        bv_s[...] = jnp.full((_B, 1), jnp.inf, jnp.float32)
        bi_s[...] = jnp.zeros((_B, 1), jnp.int32)

    r = res_ref[...]
    cb = cb_ref[0]
    cn = jnp.sum(cb * cb, axis=1)[None, :]
    d2 = (rn_s[...] - 2.0 * _dotT(r, cb)) + cn
    lv = jnp.min(d2, axis=1, keepdims=True)
    la = jnp.argmin(d2, axis=1).astype(jnp.int32)[:, None] + j * _KT
    upd = lv < bv_s[...]
    bv_s[...] = jnp.where(upd, lv, bv_s[...])
    bi_s[...] = jnp.where(upd, la, bi_s[...])

    @pl.when(j == pl.num_programs(0) - 1)
    def _():
        idxl_ref[...] = bi_s[...]
        idxg_ref[...] = bi_s[...] + (level * _K)


def _dist_call(level, resprev, qprev, codebooks):
    return pl.pallas_call(
        functools.partial(_dist_body, level),
        grid=(_K // _KT,),
        in_specs=[
            pl.BlockSpec((_B, _D), lambda j: (0, 0)),
            pl.BlockSpec((_B, _D), lambda j: (0, 0)),
            pl.BlockSpec((1, _KT, _D), lambda j, l=level: (l, j, 0)),
        ],
        out_specs=[
            pl.BlockSpec((_B, 1), lambda j: (0, 0)),
            pl.BlockSpec((_B, 1), lambda j: (0, 0)),
            pl.BlockSpec((_B, _D), lambda j: (0, 0)),
        ],
        out_shape=[
            jax.ShapeDtypeStruct((_B, 1), jnp.int32),
            jax.ShapeDtypeStruct((_B, 1), jnp.int32),
            jax.ShapeDtypeStruct((_B, _D), jnp.float32),
        ],
        scratch_shapes=[
            pltpu.VMEM((_B, 1), jnp.float32),
            pltpu.VMEM((_B, 1), jnp.float32),
            pltpu.VMEM((_B, 1), jnp.int32),
        ],
    )(resprev, qprev, codebooks)


@functools.cache
def _build_sc_gather():
    info = plsc.get_sparse_core_info()
    nc, ns = info.num_cores, info.num_subcores
    nw = nc * ns
    b_per_w = _B // nw
    mesh = plsc.VectorSubcoreMesh(core_axis_name="c", subcore_axis_name="s")

    @functools.partial(
        pl.kernel, mesh=mesh,
        out_type=jax.ShapeDtypeStruct((_B, _D), jnp.float32),
        scratch_types=[
            pltpu.VMEM((b_per_w,), jnp.int32),
            pltpu.VMEM((b_per_w, _D), jnp.float32),
            pltpu.SemaphoreType.DMA,
        ],
    )
    def gather_k(table_hbm, idx_hbm, out_hbm, idx_v, rows_v, sem):
        wid = lax.axis_index("s") * nc + lax.axis_index("c")
        base = wid * b_per_w
        pltpu.sync_copy(idx_hbm.at[pl.ds(base, b_per_w)], idx_v)
        pltpu.async_copy(table_hbm.at[idx_v], rows_v, sem).wait()
        pltpu.sync_copy(rows_v, out_hbm.at[pl.ds(base, b_per_w)])

    return gather_k


def _sc_gather(cb2d, idxg):
    return _build_sc_gather()(cb2d, idxg)


def _fin_body(x_ref, q0_ref, q1_ref, q2_ref, q3_ref, out_ref):
    qout = ((q0_ref[...] + q1_ref[...]) + q2_ref[...]) + q3_ref[...]
    out_ref[...] = x_ref[...] + (qout - x_ref[...])


def _fin_call(x, qs):
    return pl.pallas_call(
        _fin_body,
        out_shape=jax.ShapeDtypeStruct((_B, _D), jnp.float32),
    )(x, *qs)


def kernel(image_features, text_features, W_ti, b_ti, W_ii, b_ii, W_c, b_c,
           W_o, b_o, W_d1, b_d1, W_d2, b_d2, codebooks):
    feats = jnp.stack([text_features, image_features])
    w_cat = jnp.concatenate([W_ti, W_ii], axis=0)
    b_cat = jnp.concatenate([b_ti, b_ii]).reshape(1, -1)

    raw = _raw_call(feats, w_cat, b_cat)
    comb, d1 = _comb_call(raw, W_c, b_c, W_d1, b_d1)
    ds = _ds_call(d1, W_d2, b_d2)
    x = _x_call(comb, W_o, b_o, ds, text_features, image_features)

    cb2d = codebooks.reshape(_L * _K, _D)
    resprev = x
    qprev = jnp.zeros((_B, _D), jnp.float32)
    idx_locals = []
    qs = []
    for l in range(_L):
        idxl, idxg, res = _dist_call(l, resprev, qprev, codebooks)
        q = _sc_gather(cb2d, idxg.reshape(_B))
        idx_locals.append(idxl)
        qs.append(q)
        resprev, qprev = res, q

    quantized = _fin_call(x, qs)
    indices = jnp.concatenate(idx_locals, axis=1)
    return quantized, indices
